# Initial kernel scaffold; baseline (speedup 1.0000x reference)
import jax
import jax.numpy as jnp
from jax.experimental import pallas as pl


def kernel(node_features, edge_features, edge_links, params):
    raise NotImplementedError("write your pallas kernel here")


# speedup vs baseline: 2.1684x; 2.1684x over previous
import functools

import jax
import jax.numpy as jnp
from jax import lax
from jax.experimental import pallas as pl
from jax.experimental.pallas import tpu as pltpu
from jax.experimental.pallas import tpu_sc as plsc

N = 50000
E = 800000
EPS = 1e-7
NBLK = 1000
EBLK = 1024
E2 = 819200
NS = 16
CH = 1024
NCHUNK = E2 // NS // CH
G = 4
W = 16
HALF = N // 2
NPH = 26000
SSTRIPE = NPH // NS
TRASHL = HALF + 256


def _enc3_body(x_ref, w1, b1, w2, b2, w3, b3, out_ref):
    h = jnp.tanh(jnp.dot(x_ref[...], w1[...],
                         preferred_element_type=jnp.float32) + b1[...])
    h = jnp.tanh(jnp.dot(h, w2[...],
                         preferred_element_type=jnp.float32) + b2[...])
    h = jnp.tanh(jnp.dot(h, w3[...],
                         preferred_element_type=jnp.float32) + b3[...])
    for g in range(G):
        out_ref[g] = h[:, W * g:W * (g + 1)]


def _encode_cm(x, layers, blk):
    rows, din = x.shape
    grid = rows // blk
    ops, wspecs = [], []
    for l in layers:
        Wm = l["W"]
        b = l["b"].reshape(1, -1)
        ops += [Wm, b]
        wspecs += [pl.BlockSpec(Wm.shape, lambda i: (0, 0)),
                   pl.BlockSpec(b.shape, lambda i: (0, 0))]
    return pl.pallas_call(
        _enc3_body,
        grid=(grid,),
        in_specs=[pl.BlockSpec((blk, din), lambda i: (i, 0))] + wspecs,
        out_specs=pl.BlockSpec((G, blk, W), lambda i: (0, i, 0)),
        out_shape=jax.ShapeDtypeStruct((G, rows, W), jnp.float32),
    )(x, *ops)


def _cat64(ref):
    return jnp.concatenate([ref[g] for g in range(G)], axis=-1)


def _mlp128(out, w0, lnw, lnb, w4):
    h = jnp.dot(out, w0[...], preferred_element_type=jnp.float32)
    mu = jnp.mean(h, axis=-1, keepdims=True)
    var = jnp.mean((h - mu) ** 2, axis=-1, keepdims=True)
    h = (h - mu) / jnp.sqrt(var + 1e-5) * lnw[...] + lnb[...]
    h = jnp.maximum(h, 0.0)
    return jnp.dot(h, w4[...], preferred_element_type=jnp.float32)


def _acc_index_map(i):
    return (0, i + (i >= HALF // NBLK), 0)


def _combine_body(ex_ref, me_ref, x_ref, w0, lnw, lnb, w4, out_ref):
    out = _cat64(me_ref) / (_cat64(ex_ref) + 1e-16) + _cat64(x_ref)
    y = _mlp128(out, w0, lnw, lnb, w4)
    for g in range(G):
        out_ref[g] = y[:, W * g:W * (g + 1)]


def _combine(exs, mes, x_cm, p):
    return pl.pallas_call(
        _combine_body,
        grid=(N // NBLK,),
        in_specs=[
            pl.BlockSpec((G, NBLK, W), _acc_index_map),
            pl.BlockSpec((G, NBLK, W), _acc_index_map),
            pl.BlockSpec((G, NBLK, W), lambda i: (0, i, 0)),
            pl.BlockSpec((64, 128), lambda i: (0, 0)),
            pl.BlockSpec((1, 128), lambda i: (0, 0)),
            pl.BlockSpec((1, 128), lambda i: (0, 0)),
            pl.BlockSpec((128, 64), lambda i: (0, 0)),
        ],
        out_specs=pl.BlockSpec((G, NBLK, W), lambda i: (0, i, 0)),
        out_shape=jax.ShapeDtypeStruct((G, N, W), jnp.float32),
    )(exs, mes, x_cm, p["W0"], p["ln_w"].reshape(1, 128),
      p["ln_b"].reshape(1, 128), p["W4"])


def _combine_head_body(ex_ref, me_ref, x_ref, w0, lnw, lnb, w4,
                       p0w, p0b, p1w, p1b, wout, bout, out_ref):
    out = _cat64(me_ref) / (_cat64(ex_ref) + 1e-16) + _cat64(x_ref)
    y = _mlp128(out, w0, lnw, lnb, w4)
    h = jnp.tanh(jnp.dot(y, p0w[...],
                         preferred_element_type=jnp.float32) + p0b[...])
    h = jnp.tanh(jnp.dot(h, p1w[...],
                         preferred_element_type=jnp.float32) + p1b[...])
    o = jnp.sum(h * wout[...], axis=-1)
    out_ref[...] = ((o + bout[0, 0]) * 30.0).reshape(-1, 1)


def _combine_head(exs, mes, x_cm, p, pi):
    full = lambda shape: pl.BlockSpec(shape, lambda i: tuple(0 for _ in shape))
    return pl.pallas_call(
        _combine_head_body,
        grid=(N // NBLK,),
        in_specs=[
            pl.BlockSpec((G, NBLK, W), _acc_index_map),
            pl.BlockSpec((G, NBLK, W), _acc_index_map),
            pl.BlockSpec((G, NBLK, W), lambda i: (0, i, 0)),
            full((64, 128)), full((1, 128)), full((1, 128)), full((128, 64)),
            full((64, 64)), full((1, 64)),
            full((64, 64)), full((1, 64)),
            full((1, 64)), full((1, 1)),
        ],
        out_specs=pl.BlockSpec((NBLK, 1), lambda i: (i, 0)),
        out_shape=jax.ShapeDtypeStruct((N, 1), jnp.float32),
    )(exs, mes, x_cm, p["W0"], p["ln_w"].reshape(1, 128),
      p["ln_b"].reshape(1, 128), p["W4"],
      pi[0]["W"], pi[0]["b"].reshape(1, 64),
      pi[1]["W"], pi[1]["b"].reshape(1, 64),
      pi[2]["W"][:, 0].reshape(1, 64), pi[2]["b"][0].reshape(1, 1))


def _sc_edge_pass(x_flat, e_flat, src2d, dst2d, zrows):
    mesh = plsc.VectorSubcoreMesh(core_axis_name="c", subcore_axis_name="s")

    @functools.partial(
        pl.kernel, mesh=mesh,
        compiler_params=pltpu.CompilerParams(use_tc_tiling_on_sc=False),
        out_type=[jax.ShapeDtypeStruct((G, 2, NPH, W), jnp.float32),
                  jax.ShapeDtypeStruct((G, 2, NPH, W), jnp.float32)],
        scratch_types=[
            pltpu.VMEM((8, 128), jnp.int32),
            pltpu.VMEM((8, 128), jnp.int32),
            pltpu.VMEM((CH, W), jnp.float32),
            pltpu.VMEM((CH, W), jnp.float32),
            pltpu.VMEM((CH, W), jnp.float32),
            pltpu.VMEM((CH, W), jnp.float32),
            pltpu.VMEM_SHARED((NPH, W), jnp.float32),
            pltpu.VMEM_SHARED((NPH, W), jnp.float32),
            pltpu.SemaphoreType.DMA,
            pltpu.SemaphoreType.DMA,
        ],
    )
    def k(x_hbm, e_hbm, src_hbm, dst_hbm, z_hbm, exo, meo,
          sidx, didx, xg, ea, exb, meb, exacc, meacc, semg, sems):
        c = lax.axis_index("c")
        s = lax.axis_index("s")
        base = c * HALF
        r0 = s * SSTRIPE

        for g in range(G):
            pltpu.sync_copy(z_hbm, exacc.at[pl.ds(r0, SSTRIPE)])
            pltpu.sync_copy(z_hbm, meacc.at[pl.ds(r0, SSTRIPE)])
            plsc.subcore_barrier()

            def chunk(kk, carry):
                row0 = s * 400 + kk * 8
                e0 = g * E2 + s * (E2 // NS) + kk * CH
                pltpu.sync_copy(src_hbm.at[pl.ds(g * (E2 // 128) + row0, 8)],
                                sidx)
                pltpu.sync_copy(dst_hbm.at[pl.ds(row0, 8)], didx)
                for j in range(8):
                    def loc(l, carry2):
                        d = didx[j, pl.ds(l * 16, 16)] - base
                        ok = (d >= 0) & (d < HALF)
                        didx[j, pl.ds(l * 16, 16)] = jnp.where(ok, d, TRASHL)
                        return carry2
                    lax.fori_loop(0, 8, loc, 0)
                cps = [pltpu.async_copy(x_hbm.at[sidx.at[j]],
                                        xg.at[pl.ds(j * 128, 128)], semg)
                       for j in range(8)]
                cps.append(pltpu.async_copy(e_hbm.at[pl.ds(e0, CH)], ea,
                                            semg))
                for cp in cps:
                    cp.wait()

                def comp(j, carry2):
                    v = xg[j] + ea[j]
                    m = jnp.maximum(v, 0.0) + EPS
                    e = jnp.exp(m)
                    exb[j] = e
                    meb[j] = m * e
                    return carry2
                lax.fori_loop(0, CH, comp, 0)

                scs = []
                for j in range(8):
                    sl = pl.ds(j * 128, 128)
                    scs.append(pltpu.async_copy(
                        exb.at[sl], exacc.at[didx.at[j]], sems, add=True))
                    scs.append(pltpu.async_copy(
                        meb.at[sl], meacc.at[didx.at[j]], sems, add=True))
                for cp in scs:
                    cp.wait()
                return carry
            lax.fori_loop(0, NCHUNK, chunk, 0)
            plsc.subcore_barrier()

            pltpu.sync_copy(exacc.at[pl.ds(r0, SSTRIPE)],
                            exo.at[g, c, pl.ds(r0, SSTRIPE)])
            pltpu.sync_copy(meacc.at[pl.ds(r0, SSTRIPE)],
                            meo.at[g, c, pl.ds(r0, SSTRIPE)])
            plsc.subcore_barrier()

    return k(x_flat, e_flat, src2d, dst2d, zrows)


def kernel(node_features, edge_features, edge_links, params):
    nf = jnp.pad(node_features, ((0, 0), (0, 5)))
    ef = jnp.pad(edge_features, ((0, E2 - E), (0, 13)))
    nlayers = [dict(l) for l in params["node_enc"]]
    nlayers[0] = {"W": jnp.pad(nlayers[0]["W"], ((0, 5), (0, 0))),
                  "b": nlayers[0]["b"]}
    elayers = [dict(l) for l in params["edge_enc"]]
    elayers[0] = {"W": jnp.pad(elayers[0]["W"], ((0, 13), (0, 0))),
                  "b": elayers[0]["b"]}

    src = edge_links[0]
    dst = edge_links[1]
    src4 = src[None, :] + jnp.arange(G, dtype=jnp.int32)[:, None] * N
    src4 = jnp.pad(src4, ((0, 0), (0, E2 - E)))
    src2d = src4.reshape(G * E2 // 128, 128)
    dstp = jnp.concatenate(
        [dst, jnp.full((E2 - E,), N, jnp.int32)])
    dst2d = dstp.reshape(E2 // 128, 128)
    zrows = jnp.zeros((SSTRIPE, W), jnp.float32)

    x_cm = _encode_cm(nf, nlayers, NBLK)
    e_cm = _encode_cm(ef, elayers, EBLK)
    e_flat = e_cm.reshape(G * E2, W)

    x = x_cm
    for li in range(3):
        exs, mes = _sc_edge_pass(x.reshape(G * N, W), e_flat, src2d, dst2d,
                                 zrows)
        exs = exs.reshape(G, 2 * NPH, W)
        mes = mes.reshape(G, 2 * NPH, W)
        if li < 2:
            x = _combine(exs, mes, x, params["mp"][li])
        else:
            out = _combine_head(exs, mes, x, params["mp"][2],
                                params["pi"]).reshape(N)
    return out
